# Initial kernel scaffold; baseline (speedup 1.0000x reference)
import jax
import jax.numpy as jnp
from jax.experimental import pallas as pl


def kernel(artists_batch, embedding_weight):
    raise NotImplementedError("write your pallas kernel here")


# speedup vs baseline: 3.2348x; 3.2348x over previous
import jax
import jax.numpy as jnp
from jax.experimental import pallas as pl
from jax.experimental.pallas import tpu as pltpu

_VOCAB = 1000
_VPAD = 1024
_DIM = 128
_HIST = 200
_BT = 512
_TC = 8
_NT = _HIST // _TC


def _body(idx_ref, ehi_ref, elo_ref, out_ref, counts_ref):
    c = pl.program_id(1)

    @pl.when(c == 0)
    def _init():
        counts_ref[...] = jnp.zeros((_BT, _VPAD), jnp.float32)

    iota = jax.lax.broadcasted_iota(jnp.int32, (1, _VPAD), 1)
    idx8 = idx_ref[...].reshape(_BT, _TC)
    acc = None
    for j in range(_TC):
        col = jax.lax.slice_in_dim(idx8, j, j + 1, axis=1)
        oh = (col == iota).astype(jnp.float32)
        acc = oh if acc is None else acc + oh
    counts_ref[...] += acc

    @pl.when(c == _NT - 1)
    def _emit():
        cb = counts_ref[...].astype(jnp.bfloat16)
        r = jnp.dot(cb, ehi_ref[...], preferred_element_type=jnp.float32)
        r = r + jnp.dot(cb, elo_ref[...], preferred_element_type=jnp.float32)
        out_ref[...] = jnp.maximum(r * (1.0 / _HIST), 0.0)


def kernel(artists_batch, embedding_weight):
    batch = artists_batch.shape[0]
    idx4 = artists_batch.reshape(batch, _NT, 1, _TC)
    ew = jnp.pad(embedding_weight, ((0, _VPAD - _VOCAB), (0, 0)))
    ehi = ew.astype(jnp.bfloat16)
    elo = (ew - ehi.astype(jnp.float32)).astype(jnp.bfloat16)
    return pl.pallas_call(
        _body,
        grid=(batch // _BT, _NT),
        in_specs=[
            pl.BlockSpec((_BT, 1, 1, _TC), lambda i, c: (i, c, 0, 0)),
            pl.BlockSpec((_VPAD, _DIM), lambda i, c: (0, 0)),
            pl.BlockSpec((_VPAD, _DIM), lambda i, c: (0, 0)),
        ],
        out_specs=pl.BlockSpec((_BT, _DIM), lambda i, c: (i, 0)),
        out_shape=jax.ShapeDtypeStruct((batch, _DIM), jnp.float32),
        scratch_shapes=[pltpu.VMEM((_BT, _VPAD), jnp.float32)],
    )(idx4, ehi, elo)


# speedup vs baseline: 58.1652x; 17.9813x over previous
import dataclasses

import jax
import jax.numpy as jnp
from jax import lax
from jax.experimental import pallas as pl
from jax.experimental.pallas import tpu as pltpu
from jax.experimental.pallas import tpu_sc as plsc

_VOCAB = 1000
_VPAD = 1024
_DIM = 128
_HIST = 200
_NC = 2
_NS = 16
_L = 16
_NW = _NC * _NS
_G = 16
_MB = 1024


def _hist_body(idx_hbm, counts_hbm, idx_v, bins_v, sem):
    del sem
    rows_per_tile = idx_hbm.shape[0] // _NW
    n_groups = rows_per_tile // _G
    wid = lax.axis_index("s") * _NC + lax.axis_index("c")
    base = wid * rows_per_tile
    rows = lax.broadcasted_iota(jnp.int32, (_L,), 0)
    ones = jnp.ones((_L,), jnp.float32)
    zeros = jnp.zeros((_L,), jnp.float32)

    @pl.loop(0, n_groups)
    def _group(g):
        r0 = base + g * _G
        pltpu.sync_copy(idx_hbm.at[pl.ds(r0, _G)], idx_v)

        @pl.loop(0, _G)
        def _zero_row(r):
            @pl.loop(0, _VPAD, step=_L, unroll=8)
            def _zero(i):
                bins_v[r, pl.ds(i, _L)] = zeros

        @pl.loop(0, _HIST, unroll=8)
        def _scatter(t):
            tv = jnp.broadcast_to(t, (_L,))
            idxs = plsc.load_gather(idx_v, [rows, tv])
            plsc.addupdate_scatter(bins_v, [rows, idxs], ones)

        pltpu.sync_copy(bins_v, counts_hbm.at[pl.ds(r0, _G)])


def _mm_body(cnt_ref, ehi_ref, elo_ref, out_ref):
    cb = cnt_ref[...].astype(jnp.bfloat16)
    r = jnp.dot(cb, ehi_ref[...], preferred_element_type=jnp.float32)
    r = r + jnp.dot(cb, elo_ref[...], preferred_element_type=jnp.float32)
    out_ref[...] = jnp.maximum(r * (1.0 / _HIST), 0.0)


def kernel(artists_batch, embedding_weight):
    batch = artists_batch.shape[0]

    sc_params = pltpu.CompilerParams()
    if "needs_layout_passes" in pltpu.CompilerParams.__dataclass_fields__:
        sc_params = dataclasses.replace(sc_params, needs_layout_passes=False)
    mesh = plsc.VectorSubcoreMesh(core_axis_name="c", subcore_axis_name="s")
    counts = pl.kernel(
        _hist_body,
        out_type=jax.ShapeDtypeStruct((batch, _VPAD), jnp.float32),
        mesh=mesh,
        scratch_types=[
            pltpu.VMEM((_G, _HIST), jnp.int32),
            pltpu.VMEM((_G, _VPAD), jnp.float32),
            pltpu.SemaphoreType.DMA,
        ],
        compiler_params=sc_params,
    )(artists_batch)

    ew = jnp.pad(embedding_weight, ((0, _VPAD - _VOCAB), (0, 0)))
    ehi = ew.astype(jnp.bfloat16)
    elo = (ew - ehi.astype(jnp.float32)).astype(jnp.bfloat16)
    return pl.pallas_call(
        _mm_body,
        grid=(batch // _MB,),
        in_specs=[
            pl.BlockSpec((_MB, _VPAD), lambda i: (i, 0)),
            pl.BlockSpec((_VPAD, _DIM), lambda i: (0, 0)),
            pl.BlockSpec((_VPAD, _DIM), lambda i: (0, 0)),
        ],
        out_specs=pl.BlockSpec((_MB, _DIM), lambda i: (i, 0)),
        out_shape=jax.ShapeDtypeStruct((batch, _DIM), jnp.float32),
    )(counts, ehi, elo)


# speedup vs baseline: 72.1601x; 1.2406x over previous
import dataclasses

import jax
import jax.numpy as jnp
from jax import lax
from jax.experimental import pallas as pl
from jax.experimental.pallas import tpu as pltpu
from jax.experimental.pallas import tpu_sc as plsc

_VOCAB = 1000
_VPAD = 1024
_DIM = 128
_HIST = 200
_NC = 2
_NS = 16
_L = 16
_NW = _NC * _NS
_G = 16
_MB = 1024


def _hist_body(idx_hbm, counts_hbm, idx_bufs, bins_bufs, sems_in, sems_out):
    rows_per_tile = idx_hbm.shape[0] // _NW
    n_groups = rows_per_tile // _G
    wid = lax.axis_index("s") * _NC + lax.axis_index("c")
    base = wid * rows_per_tile
    rows = lax.broadcasted_iota(jnp.int32, (_L,), 0)
    ones = jnp.ones((_L,), jnp.float32)
    zeros = jnp.zeros((_L,), jnp.float32)

    def row0(g):
        return base + g * _G

    pltpu.async_copy(idx_hbm.at[pl.ds(row0(0), _G)], idx_bufs[0], sems_in[0])
    pltpu.async_copy(idx_hbm.at[pl.ds(row0(1), _G)], idx_bufs[1], sems_in[1])

    def body(g, ib, bb):
        idx_v, sem_in = idx_bufs[ib], sems_in[ib]
        bins_v, sem_out = bins_bufs[bb], sems_out[bb]
        ib_next = (ib + 2) % 4
        pltpu.make_async_copy(
            idx_hbm.at[pl.ds(row0(g), _G)], idx_v, sem_in
        ).wait()

        @pl.when(g + 2 < n_groups)
        def _prefetch():
            pltpu.async_copy(
                idx_hbm.at[pl.ds(row0(g + 2), _G)],
                idx_bufs[ib_next],
                sems_in[ib_next],
            )

        @pl.when(g >= 2)
        def _wait_out():
            pltpu.make_async_copy(
                bins_v, counts_hbm.at[pl.ds(row0(g - 2), _G)], sem_out
            ).wait()

        @pl.loop(0, _G)
        def _zero_row(r):
            @pl.loop(0, _VPAD, step=_L, unroll=8)
            def _zero(i):
                bins_v[r, pl.ds(i, _L)] = zeros

        @pl.loop(0, _HIST, unroll=8)
        def _scatter(t):
            tv = jnp.broadcast_to(t, (_L,))
            idxs = plsc.load_gather(idx_v, [rows, tv])
            plsc.addupdate_scatter(bins_v, [rows, idxs], ones)

        pltpu.async_copy(bins_v, counts_hbm.at[pl.ds(row0(g), _G)], sem_out)

    @pl.loop(0, n_groups, step=4)
    def _group(g):
        body(g, 0, 0)
        body(g + 1, 1, 1)
        body(g + 2, 2, 0)
        body(g + 3, 3, 1)

    pltpu.make_async_copy(
        bins_bufs[0], counts_hbm.at[pl.ds(row0(n_groups - 2), _G)], sems_out[0]
    ).wait()
    pltpu.make_async_copy(
        bins_bufs[1], counts_hbm.at[pl.ds(row0(n_groups - 1), _G)], sems_out[1]
    ).wait()


def _mm_body(cnt_ref, ehi_ref, elo_ref, out_ref):
    cb = cnt_ref[...].astype(jnp.bfloat16)
    r = jnp.dot(cb, ehi_ref[...], preferred_element_type=jnp.float32)
    r = r + jnp.dot(cb, elo_ref[...], preferred_element_type=jnp.float32)
    out_ref[...] = jnp.maximum(r * (1.0 / _HIST), 0.0)


def kernel(artists_batch, embedding_weight):
    batch = artists_batch.shape[0]

    sc_params = pltpu.CompilerParams()
    if "needs_layout_passes" in pltpu.CompilerParams.__dataclass_fields__:
        sc_params = dataclasses.replace(sc_params, needs_layout_passes=False)
    mesh = plsc.VectorSubcoreMesh(core_axis_name="c", subcore_axis_name="s")
    counts = pl.kernel(
        _hist_body,
        out_type=jax.ShapeDtypeStruct((batch, _VPAD), jnp.float32),
        mesh=mesh,
        scratch_types=[
            [pltpu.VMEM((_G, _HIST), jnp.int32) for _ in range(4)],
            [pltpu.VMEM((_G, _VPAD), jnp.float32) for _ in range(2)],
            [pltpu.SemaphoreType.DMA for _ in range(4)],
            [pltpu.SemaphoreType.DMA for _ in range(2)],
        ],
        compiler_params=sc_params,
    )(artists_batch)

    ew = jnp.pad(embedding_weight, ((0, _VPAD - _VOCAB), (0, 0)))
    ehi = ew.astype(jnp.bfloat16)
    elo = (ew - ehi.astype(jnp.float32)).astype(jnp.bfloat16)
    return pl.pallas_call(
        _mm_body,
        grid=(batch // _MB,),
        in_specs=[
            pl.BlockSpec((_MB, _VPAD), lambda i: (i, 0)),
            pl.BlockSpec((_VPAD, _DIM), lambda i: (0, 0)),
            pl.BlockSpec((_VPAD, _DIM), lambda i: (0, 0)),
        ],
        out_specs=pl.BlockSpec((_MB, _DIM), lambda i: (i, 0)),
        out_shape=jax.ShapeDtypeStruct((batch, _DIM), jnp.float32),
    )(counts, ehi, elo)
